# baseline (device time: 134302 ns/iter reference)
import jax
import jax.numpy as jnp
from jax import lax
from jax.experimental import pallas as pl
from jax.experimental.pallas import tpu as pltpu

N_DEV = 16
N_RINGS = 4
BAND = 2048 // N_RINGS
CH = BAND // N_DEV
N_HOP = N_DEV - 1


def kernel(t):
    m, n = t.shape

    def body(x_ref, out_ref, partial_ref, recv_sems, send_sems):
        my = lax.axis_index("i")
        left = (my + N_DEV - 1) % N_DEV
        right = (my + 1) % N_DEV

        barrier_sem = pltpu.get_barrier_semaphore()
        for nbr in (left, right):
            pl.semaphore_signal(
                barrier_sem, inc=1,
                device_id=(nbr,), device_id_type=pl.DeviceIdType.MESH,
            )
        pl.semaphore_wait(barrier_sem, 2)

        def rightward(r):
            return r < N_RINGS // 2

        def dest(r):
            return right if rightward(r) else left

        def rs_send_c(r, h):
            return (my + (N_DEV - h if rightward(r) else h)) % N_DEV

        def rs_recv_c(r, h):
            return (my + (N_DEV - h - 1 if rightward(r) else h + 1)) % N_DEV

        def ag_send_c(r, g):
            return (my + (N_DEV + 1 - g if rightward(r) else N_DEV - 1 + g)) % N_DEV

        def row0(r, c):
            return r * BAND + c * CH

        def chunk_slice(ref, r, c):
            return ref.at[pl.ds(row0(r, c), CH), :]

        def make(r, h, src_ref, dst_ref):
            return pltpu.make_async_remote_copy(
                src_ref=src_ref,
                dst_ref=dst_ref,
                send_sem=send_sems.at[r, h % 2],
                recv_sem=recv_sems.at[r, h],
                device_id=(dest(r),),
                device_id_type=pl.DeviceIdType.MESH,
            )

        desc = [[None] * N_HOP for _ in range(N_RINGS)]
        for r in range(N_RINGS):
            d = make(r, 0, chunk_slice(x_ref, r, rs_send_c(r, 0)),
                     partial_ref.at[r, 0])
            d.start()
            desc[r][0] = d

        for h in range(1, N_HOP):
            for r in range(N_RINGS):
                desc[r][h - 1].wait_recv()
                c = rs_recv_c(r, h - 1)
                rs = row0(r, c)
                out_ref[pl.ds(rs, CH), :] = (
                    x_ref[pl.ds(rs, CH), :] + partial_ref[r, h - 1]
                )
                if h >= 2:
                    desc[r][h - 2].wait_send()
                d = make(r, h, chunk_slice(out_ref, r, c), partial_ref.at[r, h])
                d.start()
                desc[r][h] = d

        for r in range(N_RINGS):
            desc[r][N_HOP - 1].wait_recv()
            own = rs_recv_c(r, N_HOP - 1)
            rs = row0(r, own)
            s = x_ref[pl.ds(rs, CH), :] + partial_ref[r, N_HOP - 1]
            rl = jnp.maximum(s, 0.0)
            out_ref[pl.ds(rs, CH), :] = jnp.tanh(s) * s * s + rl * rl * rl
        for r in range(N_RINGS):
            desc[r][N_HOP - 2].wait_send()
            desc[r][N_HOP - 1].wait_send()

        ag = [[None] * N_HOP for _ in range(N_RINGS)]
        for g in range(N_HOP):
            for r in range(N_RINGS):
                if g >= 1:
                    ag[r][g - 1].wait_recv()
                if g >= 2:
                    ag[r][g - 2].wait_send()
                sc = ag_send_c(r, g)
                d = make(r, g, chunk_slice(out_ref, r, sc),
                         chunk_slice(out_ref, r, sc))
                d.start()
                ag[r][g] = d
        for r in range(N_RINGS):
            ag[r][N_HOP - 1].wait_recv()
            ag[r][N_HOP - 2].wait_send()
            ag[r][N_HOP - 1].wait_send()

    return pl.pallas_call(
        body,
        out_shape=jax.ShapeDtypeStruct((m, n), jnp.float32),
        in_specs=[pl.BlockSpec(memory_space=pltpu.VMEM)],
        out_specs=pl.BlockSpec(memory_space=pltpu.VMEM),
        scratch_shapes=[
            pltpu.VMEM((N_RINGS, N_HOP, CH, n), jnp.float32),
            pltpu.SemaphoreType.DMA((N_RINGS, N_HOP)),
            pltpu.SemaphoreType.DMA((N_RINGS, 2)),
        ],
        compiler_params=pltpu.CompilerParams(collective_id=0),
    )(t)


# device time: 111002 ns/iter; 1.2099x vs baseline; 1.2099x over previous
import jax
import jax.numpy as jnp
from jax import lax
from jax.experimental import pallas as pl
from jax.experimental.pallas import tpu as pltpu

N_DEV = 16
N_RINGS = 8
BAND = 2048 // N_RINGS
CH = BAND // N_DEV
N_HOP = N_DEV - 1

_RING = [0, 1, 5, 9, 13, 14, 10, 6, 2, 3, 7, 11, 15, 12, 8, 4]
_POS = [0] * N_DEV
for _k, _lid in enumerate(_RING):
    _POS[_lid] = _k


def kernel(t):
    m, n = t.shape

    def body(scal_ref, x_ref, out_ref, partial_ref, recv_sems, send_sems):
        my = scal_ref[0]
        right = scal_ref[1]
        left = scal_ref[2]

        barrier_sem = pltpu.get_barrier_semaphore()
        for nbr in (left, right):
            pl.semaphore_signal(
                barrier_sem, inc=1,
                device_id=(nbr,), device_id_type=pl.DeviceIdType.MESH,
            )
        pl.semaphore_wait(barrier_sem, 2)

        def rightward(r):
            return r < N_RINGS // 2

        def dest(r):
            return right if rightward(r) else left

        def rs_send_c(r, h):
            return (my + (N_DEV - h if rightward(r) else h)) % N_DEV

        def rs_recv_c(r, h):
            return (my + (N_DEV - h - 1 if rightward(r) else h + 1)) % N_DEV

        def ag_send_c(r, g):
            return (my + (N_DEV + 1 - g if rightward(r) else N_DEV - 1 + g)) % N_DEV

        def row0(r, c):
            return r * BAND + c * CH

        def chunk_slice(ref, r, c):
            return ref.at[pl.ds(row0(r, c), CH), :]

        def make(r, h, src_ref, dst_ref):
            return pltpu.make_async_remote_copy(
                src_ref=src_ref,
                dst_ref=dst_ref,
                send_sem=send_sems.at[r, h % 2],
                recv_sem=recv_sems.at[r, h],
                device_id=(dest(r),),
                device_id_type=pl.DeviceIdType.MESH,
            )

        desc = [[None] * N_HOP for _ in range(N_RINGS)]
        for r in range(N_RINGS):
            d = make(r, 0, chunk_slice(x_ref, r, rs_send_c(r, 0)),
                     partial_ref.at[r, 0])
            d.start()
            desc[r][0] = d

        for h in range(1, N_HOP):
            for r in range(N_RINGS):
                desc[r][h - 1].wait_recv()
                c = rs_recv_c(r, h - 1)
                rs = row0(r, c)
                out_ref[pl.ds(rs, CH), :] = (
                    x_ref[pl.ds(rs, CH), :] + partial_ref[r, h - 1]
                )
                if h >= 2:
                    desc[r][h - 2].wait_send()
                d = make(r, h, chunk_slice(out_ref, r, c), partial_ref.at[r, h])
                d.start()
                desc[r][h] = d

        for r in range(N_RINGS):
            desc[r][N_HOP - 1].wait_recv()
            own = rs_recv_c(r, N_HOP - 1)
            rs = row0(r, own)
            s = x_ref[pl.ds(rs, CH), :] + partial_ref[r, N_HOP - 1]
            rl = jnp.maximum(s, 0.0)
            out_ref[pl.ds(rs, CH), :] = jnp.tanh(s) * s * s + rl * rl * rl
        for r in range(N_RINGS):
            desc[r][N_HOP - 2].wait_send()
            desc[r][N_HOP - 1].wait_send()

        ag = [[None] * N_HOP for _ in range(N_RINGS)]
        for g in range(N_HOP):
            for r in range(N_RINGS):
                if g >= 1:
                    ag[r][g - 1].wait_recv()
                if g >= 2:
                    ag[r][g - 2].wait_send()
                sc = ag_send_c(r, g)
                d = make(r, g, chunk_slice(out_ref, r, sc),
                         chunk_slice(out_ref, r, sc))
                d.start()
                ag[r][g] = d
        for r in range(N_RINGS):
            ag[r][N_HOP - 1].wait_recv()
            ag[r][N_HOP - 2].wait_send()
            ag[r][N_HOP - 1].wait_send()

    lid = lax.axis_index("i")
    ring = jnp.asarray(_RING, dtype=jnp.int32)
    pos = jnp.asarray(_POS, dtype=jnp.int32)[lid]
    scalars = jnp.stack([
        pos,
        ring[(pos + 1) % N_DEV],
        ring[(pos + N_DEV - 1) % N_DEV],
    ]).astype(jnp.int32)

    return pl.pallas_call(
        body,
        out_shape=jax.ShapeDtypeStruct((m, n), jnp.float32),
        in_specs=[
            pl.BlockSpec(memory_space=pltpu.SMEM),
            pl.BlockSpec(memory_space=pltpu.VMEM),
        ],
        out_specs=pl.BlockSpec(memory_space=pltpu.VMEM),
        scratch_shapes=[
            pltpu.VMEM((N_RINGS, N_HOP, CH, n), jnp.float32),
            pltpu.SemaphoreType.DMA((N_RINGS, N_HOP)),
            pltpu.SemaphoreType.DMA((N_RINGS, 2)),
        ],
        compiler_params=pltpu.CompilerParams(collective_id=0),
    )(scalars, t)
